# Initial kernel scaffold; baseline (speedup 1.0000x reference)
import jax
import jax.numpy as jnp
from jax.experimental import pallas as pl


def kernel(ego_mask_batch, big_batch_positions, big_batched_adjacency_pruned, gcn1_W, gcn1_b, gcn2_W, gcn2_b, lstm_W_ih, lstm_W_hh, lstm_b_ih, lstm_b_hh, attn_W, attn_b):
    raise NotImplementedError("write your pallas kernel here")


# speedup vs baseline: 5.1254x; 5.1254x over previous
import functools

import jax
import jax.numpy as jnp
from jax.experimental import pallas as pl
from jax.experimental.pallas import tpu as pltpu


def _gcn_step(a_ref, x_ref, m_ref, w1_ref, b1_ref, w2_ref, b2_ref, ph_ref):
    af = a_ref[0].astype(jnp.float32)
    x = x_ref[0]
    m_row = m_ref[0]
    m_col = jnp.transpose(m_row)

    dn = (((0,), (0,)), ((), ()))
    t1 = jax.lax.dot_general(af, m_col, dn, preferred_element_type=jnp.float32)
    deg = m_col * (t1 + 1.0)
    dinv = jax.lax.rsqrt(jnp.maximum(deg, 1e-12))
    md = m_col * dinv
    mdd = md * dinv

    hp1 = jnp.dot(x, w1_ref[...], preferred_element_type=jnp.float32)
    s1 = jax.lax.dot_general(af, md * hp1, dn, preferred_element_type=jnp.float32)
    h1 = jax.nn.relu(md * s1 + mdd * hp1 + b1_ref[...])

    hp2 = jnp.dot(h1, w2_ref[...], preferred_element_type=jnp.float32)
    s2 = jax.lax.dot_general(af, md * hp2, dn, preferred_element_type=jnp.float32)
    ph_ref[0] = m_col * (md * s2 + mdd * hp2 + b2_ref[...])


def _lstm_attn_step(ph_ref, wih_ref, whh_ref, b_ref, aw_ref, out_ref,
                    h_scr, c_scr, *, chunk, hid):
    @pl.when(pl.program_id(0) == 0)
    def _():
        h_scr[...] = jnp.zeros_like(h_scr)
        c_scr[...] = jnp.zeros_like(c_scr)

    wih = wih_ref[...]
    whh = whh_ref[...]
    b = b_ref[...]
    aw = aw_ref[...]

    def body(n, hc):
        h, c = hc
        xr = ph_ref[pl.ds(n, 1)].reshape(ph_ref.shape[1], hid)
        g = (jnp.dot(xr, wih, preferred_element_type=jnp.float32)
             + jnp.dot(h, whh, preferred_element_type=jnp.float32) + b)
        i = jax.nn.sigmoid(g[:, 0:hid])
        f = jax.nn.sigmoid(g[:, hid:2 * hid])
        gg = jnp.tanh(g[:, 2 * hid:3 * hid])
        o = jax.nn.sigmoid(g[:, 3 * hid:4 * hid])
        c = f * c + i * gg
        h = o * jnp.tanh(c)
        s = jnp.sum(h * aw, axis=1, keepdims=True)
        e = jnp.exp(s - jnp.max(s, axis=0, keepdims=True))
        w = e / jnp.sum(e, axis=0, keepdims=True)
        out_ref[pl.ds(n, 1), :] = jnp.sum(h * w, axis=0, keepdims=True)
        return (h, c)

    h, c = jax.lax.fori_loop(0, chunk, body, (h_scr[...], c_scr[...]))
    h_scr[...] = h
    c_scr[...] = c


def kernel(ego_mask_batch, big_batch_positions, big_batched_adjacency_pruned,
           gcn1_W, gcn1_b, gcn2_W, gcn2_b,
           lstm_W_ih, lstm_W_hh, lstm_b_ih, lstm_b_hh, attn_W, attn_b):
    T, N, F = big_batch_positions.shape
    bsz, _, max_nodes = ego_mask_batch.shape
    hid = gcn1_W.shape[1]
    G = lstm_W_ih.shape[0]

    mask = (jnp.transpose(ego_mask_batch, (1, 0, 2))
            .reshape(T, 1, N).astype(jnp.float32))

    ph = pl.pallas_call(
        _gcn_step,
        grid=(T,),
        in_specs=[
            pl.BlockSpec((1, N, N), lambda t: (t, 0, 0)),
            pl.BlockSpec((1, N, F), lambda t: (t, 0, 0)),
            pl.BlockSpec((1, 1, N), lambda t: (t, 0, 0)),
            pl.BlockSpec((F, hid), lambda t: (0, 0)),
            pl.BlockSpec((1, hid), lambda t: (0, 0)),
            pl.BlockSpec((hid, hid), lambda t: (0, 0)),
            pl.BlockSpec((1, hid), lambda t: (0, 0)),
        ],
        out_specs=pl.BlockSpec((1, N, hid), lambda t: (t, 0, 0)),
        out_shape=jax.ShapeDtypeStruct((T, N, hid), jnp.float32),
        compiler_params=pltpu.CompilerParams(
            dimension_semantics=("arbitrary",)),
    )(big_batched_adjacency_pruned, big_batch_positions, mask,
      gcn1_W, gcn1_b.reshape(1, hid), gcn2_W, gcn2_b.reshape(1, hid))

    ph_nt = jnp.transpose(ph, (1, 0, 2))

    chunk = 256
    bias = (lstm_b_ih + lstm_b_hh).reshape(1, G)
    xout = pl.pallas_call(
        functools.partial(_lstm_attn_step, chunk=chunk, hid=hid),
        grid=(N // chunk,),
        in_specs=[
            pl.BlockSpec((chunk, T, hid), lambda i: (i, 0, 0)),
            pl.BlockSpec((hid, G), lambda i: (0, 0)),
            pl.BlockSpec((hid, G), lambda i: (0, 0)),
            pl.BlockSpec((1, G), lambda i: (0, 0)),
            pl.BlockSpec((1, hid), lambda i: (0, 0)),
        ],
        out_specs=pl.BlockSpec((chunk, hid), lambda i: (i, 0)),
        out_shape=jax.ShapeDtypeStruct((N, hid), jnp.float32),
        scratch_shapes=[
            pltpu.VMEM((T, hid), jnp.float32),
            pltpu.VMEM((T, hid), jnp.float32),
        ],
        compiler_params=pltpu.CompilerParams(
            dimension_semantics=("arbitrary",)),
    )(ph_nt, jnp.transpose(lstm_W_ih), jnp.transpose(lstm_W_hh), bias, attn_W)

    return xout.reshape(bsz, max_nodes, hid)


# speedup vs baseline: 6.9021x; 1.3467x over previous
import functools

import jax
import jax.numpy as jnp
from jax.experimental import pallas as pl
from jax.experimental.pallas import tpu as pltpu


def _gcn_step(a_ref, x_ref, m_ref, w1_ref, b1_ref, w2_ref, b2_ref, ph_ref):
    af = a_ref[0].astype(jnp.bfloat16)
    x = x_ref[0]
    m_row = m_ref[0]
    m_col = jnp.transpose(m_row)

    dn = (((0,), (0,)), ((), ()))
    t1 = jax.lax.dot_general(af, m_col.astype(jnp.bfloat16), dn,
                             preferred_element_type=jnp.float32)
    deg = m_col * (t1 + 1.0)
    dinv = jax.lax.rsqrt(jnp.maximum(deg, 1e-12))
    md = m_col * dinv
    mdd = md * dinv

    hp1 = jnp.dot(x, w1_ref[...], preferred_element_type=jnp.float32)
    s1 = jax.lax.dot_general(af, (md * hp1).astype(jnp.bfloat16), dn,
                             preferred_element_type=jnp.float32)
    h1 = jax.nn.relu(md * s1 + mdd * hp1 + b1_ref[...])

    hp2 = jnp.dot(h1, w2_ref[...], preferred_element_type=jnp.float32)
    s2 = jax.lax.dot_general(af, (md * hp2).astype(jnp.bfloat16), dn,
                             preferred_element_type=jnp.float32)
    ph_ref[0] = m_col * (md * s2 + mdd * hp2 + b2_ref[...])


def _lstm_attn_step(ph_ref, wih_ref, whh_ref, b_ref, aw_ref, out_ref,
                    h_scr, c_scr, *, chunk, hid):
    @pl.when(pl.program_id(0) == 0)
    def _():
        h_scr[...] = jnp.zeros_like(h_scr)
        c_scr[...] = jnp.zeros_like(c_scr)

    wih = wih_ref[...]
    whh = whh_ref[...]
    b = b_ref[...]
    aw = aw_ref[...]

    def body(n, hc):
        h, c = hc
        xr = ph_ref[pl.ds(n, 1)].reshape(ph_ref.shape[1], hid)
        g = (jnp.dot(xr, wih, preferred_element_type=jnp.float32)
             + jnp.dot(h, whh, preferred_element_type=jnp.float32) + b)
        i = jax.nn.sigmoid(g[:, 0:hid])
        f = jax.nn.sigmoid(g[:, hid:2 * hid])
        gg = jnp.tanh(g[:, 2 * hid:3 * hid])
        o = jax.nn.sigmoid(g[:, 3 * hid:4 * hid])
        c = f * c + i * gg
        h = o * jnp.tanh(c)
        s = jnp.sum(h * aw, axis=1, keepdims=True)
        e = jnp.exp(s - jnp.max(s, axis=0, keepdims=True))
        w = e / jnp.sum(e, axis=0, keepdims=True)
        out_ref[pl.ds(n, 1), :] = jnp.sum(h * w, axis=0, keepdims=True)
        return (h, c)

    h, c = jax.lax.fori_loop(0, chunk, body, (h_scr[...], c_scr[...]),
                             unroll=4)
    h_scr[...] = h
    c_scr[...] = c


def kernel(ego_mask_batch, big_batch_positions, big_batched_adjacency_pruned,
           gcn1_W, gcn1_b, gcn2_W, gcn2_b,
           lstm_W_ih, lstm_W_hh, lstm_b_ih, lstm_b_hh, attn_W, attn_b):
    T, N, F = big_batch_positions.shape
    bsz, _, max_nodes = ego_mask_batch.shape
    hid = gcn1_W.shape[1]
    G = lstm_W_ih.shape[0]

    mask = (jnp.transpose(ego_mask_batch, (1, 0, 2))
            .reshape(T, 1, N).astype(jnp.float32))

    ph = pl.pallas_call(
        _gcn_step,
        grid=(T,),
        in_specs=[
            pl.BlockSpec((1, N, N), lambda t: (t, 0, 0)),
            pl.BlockSpec((1, N, F), lambda t: (t, 0, 0)),
            pl.BlockSpec((1, 1, N), lambda t: (t, 0, 0)),
            pl.BlockSpec((F, hid), lambda t: (0, 0)),
            pl.BlockSpec((1, hid), lambda t: (0, 0)),
            pl.BlockSpec((hid, hid), lambda t: (0, 0)),
            pl.BlockSpec((1, hid), lambda t: (0, 0)),
        ],
        out_specs=pl.BlockSpec((1, N, hid), lambda t: (t, 0, 0)),
        out_shape=jax.ShapeDtypeStruct((T, N, hid), jnp.float32),
        compiler_params=pltpu.CompilerParams(
            dimension_semantics=("arbitrary",)),
    )(big_batched_adjacency_pruned, big_batch_positions, mask,
      gcn1_W, gcn1_b.reshape(1, hid), gcn2_W, gcn2_b.reshape(1, hid))

    ph_nt = jnp.transpose(ph, (1, 0, 2))

    chunk = 256
    bias = (lstm_b_ih + lstm_b_hh).reshape(1, G)
    xout = pl.pallas_call(
        functools.partial(_lstm_attn_step, chunk=chunk, hid=hid),
        grid=(N // chunk,),
        in_specs=[
            pl.BlockSpec((chunk, T, hid), lambda i: (i, 0, 0)),
            pl.BlockSpec((hid, G), lambda i: (0, 0)),
            pl.BlockSpec((hid, G), lambda i: (0, 0)),
            pl.BlockSpec((1, G), lambda i: (0, 0)),
            pl.BlockSpec((1, hid), lambda i: (0, 0)),
        ],
        out_specs=pl.BlockSpec((chunk, hid), lambda i: (i, 0)),
        out_shape=jax.ShapeDtypeStruct((N, hid), jnp.float32),
        scratch_shapes=[
            pltpu.VMEM((T, hid), jnp.float32),
            pltpu.VMEM((T, hid), jnp.float32),
        ],
        compiler_params=pltpu.CompilerParams(
            dimension_semantics=("arbitrary",)),
    )(ph_nt, jnp.transpose(lstm_W_ih), jnp.transpose(lstm_W_hh), bias, attn_W)

    return xout.reshape(bsz, max_nodes, hid)


# speedup vs baseline: 7.0306x; 1.0186x over previous
import functools

import jax
import jax.numpy as jnp
from jax.experimental import pallas as pl
from jax.experimental.pallas import tpu as pltpu


def _gcn_step(a_ref, x_ref, m_ref, w1_ref, b1_ref, w2_ref, b2_ref, ph_ref):
    af = a_ref[0].astype(jnp.bfloat16)
    x = x_ref[0]
    m_row = m_ref[0]
    m_col = jnp.transpose(m_row)

    dn = (((0,), (0,)), ((), ()))
    t1 = jax.lax.dot_general(af, m_col.astype(jnp.bfloat16), dn,
                             preferred_element_type=jnp.float32)
    deg = m_col * (t1 + 1.0)
    dinv = jax.lax.rsqrt(jnp.maximum(deg, 1e-12))
    md = m_col * dinv
    mdd = md * dinv

    hp1 = jnp.dot(x, w1_ref[...], preferred_element_type=jnp.float32)
    s1 = jax.lax.dot_general(af, (md * hp1).astype(jnp.bfloat16), dn,
                             preferred_element_type=jnp.float32)
    h1 = jax.nn.relu(md * s1 + mdd * hp1 + b1_ref[...])

    hp2 = jnp.dot(h1, w2_ref[...], preferred_element_type=jnp.float32)
    s2 = jax.lax.dot_general(af, (md * hp2).astype(jnp.bfloat16), dn,
                             preferred_element_type=jnp.float32)
    ph_ref[0] = m_col * (md * s2 + mdd * hp2 + b2_ref[...])


def _lstm_attn_step(ph_ref, wih_ref, whh_ref, b_ref, aw_ref, out_ref,
                    h_scr, c_scr, *, chunk, hid):
    @pl.when(pl.program_id(0) == 0)
    def _():
        h_scr[...] = jnp.zeros_like(h_scr)
        c_scr[...] = jnp.zeros_like(c_scr)

    wih = wih_ref[...].astype(jnp.bfloat16)
    whh = whh_ref[...].astype(jnp.bfloat16)
    b = b_ref[...]
    aw = aw_ref[...]

    def body(n, hc):
        h, c = hc
        xr = ph_ref[pl.ds(n, 1)].reshape(ph_ref.shape[1], hid)
        g = (jnp.dot(xr.astype(jnp.bfloat16), wih,
                     preferred_element_type=jnp.float32)
             + jnp.dot(h.astype(jnp.bfloat16), whh,
                       preferred_element_type=jnp.float32) + b)
        i = jax.nn.sigmoid(g[:, 0:hid])
        f = jax.nn.sigmoid(g[:, hid:2 * hid])
        gg = jnp.tanh(g[:, 2 * hid:3 * hid])
        o = jax.nn.sigmoid(g[:, 3 * hid:4 * hid])
        c = f * c + i * gg
        h = o * jnp.tanh(c)
        s = jnp.sum(h * aw, axis=1, keepdims=True)
        e = jnp.exp(s - jnp.max(s, axis=0, keepdims=True))
        w = e / jnp.sum(e, axis=0, keepdims=True)
        out_ref[pl.ds(n, 1), :] = jnp.sum(h * w, axis=0, keepdims=True)
        return (h, c)

    h, c = jax.lax.fori_loop(0, chunk, body, (h_scr[...], c_scr[...]),
                             unroll=8)
    h_scr[...] = h
    c_scr[...] = c


def kernel(ego_mask_batch, big_batch_positions, big_batched_adjacency_pruned,
           gcn1_W, gcn1_b, gcn2_W, gcn2_b,
           lstm_W_ih, lstm_W_hh, lstm_b_ih, lstm_b_hh, attn_W, attn_b):
    T, N, F = big_batch_positions.shape
    bsz, _, max_nodes = ego_mask_batch.shape
    hid = gcn1_W.shape[1]
    G = lstm_W_ih.shape[0]

    mask = (jnp.transpose(ego_mask_batch, (1, 0, 2))
            .reshape(T, 1, N).astype(jnp.float32))

    ph = pl.pallas_call(
        _gcn_step,
        grid=(T,),
        in_specs=[
            pl.BlockSpec((1, N, N), lambda t: (t, 0, 0)),
            pl.BlockSpec((1, N, F), lambda t: (t, 0, 0)),
            pl.BlockSpec((1, 1, N), lambda t: (t, 0, 0)),
            pl.BlockSpec((F, hid), lambda t: (0, 0)),
            pl.BlockSpec((1, hid), lambda t: (0, 0)),
            pl.BlockSpec((hid, hid), lambda t: (0, 0)),
            pl.BlockSpec((1, hid), lambda t: (0, 0)),
        ],
        out_specs=pl.BlockSpec((1, N, hid), lambda t: (t, 0, 0)),
        out_shape=jax.ShapeDtypeStruct((T, N, hid), jnp.float32),
        compiler_params=pltpu.CompilerParams(
            dimension_semantics=("arbitrary",)),
    )(big_batched_adjacency_pruned, big_batch_positions, mask,
      gcn1_W, gcn1_b.reshape(1, hid), gcn2_W, gcn2_b.reshape(1, hid))

    ph_nt = jnp.transpose(ph, (1, 0, 2))

    chunk = 256
    bias = (lstm_b_ih + lstm_b_hh).reshape(1, G)
    xout = pl.pallas_call(
        functools.partial(_lstm_attn_step, chunk=chunk, hid=hid),
        grid=(N // chunk,),
        in_specs=[
            pl.BlockSpec((chunk, T, hid), lambda i: (i, 0, 0)),
            pl.BlockSpec((hid, G), lambda i: (0, 0)),
            pl.BlockSpec((hid, G), lambda i: (0, 0)),
            pl.BlockSpec((1, G), lambda i: (0, 0)),
            pl.BlockSpec((1, hid), lambda i: (0, 0)),
        ],
        out_specs=pl.BlockSpec((chunk, hid), lambda i: (i, 0)),
        out_shape=jax.ShapeDtypeStruct((N, hid), jnp.float32),
        scratch_shapes=[
            pltpu.VMEM((T, hid), jnp.float32),
            pltpu.VMEM((T, hid), jnp.float32),
        ],
        compiler_params=pltpu.CompilerParams(
            dimension_semantics=("arbitrary",)),
    )(ph_nt, jnp.transpose(lstm_W_ih), jnp.transpose(lstm_W_hh), bias, attn_W)

    return xout.reshape(bsz, max_nodes, hid)
